# baseline (speedup 1.0000x reference)
import functools

import jax
import jax.numpy as jnp
from jax import lax
from jax.experimental import pallas as pl
from jax.experimental.pallas import tpu as pltpu
from jax.experimental.pallas import tpu_sc as plsc

B = 8
C = 96
H = 384
NPIX = 384 * 384
K = int(NPIX * 0.1)
NB = 32768
BH = 16
NSPLIT = 4
HPART = H // NSPLIT
PPIX = NPIX // NSPLIT
NCORES = 2
NSUB = 16
TPIX = PPIX // 4
CHUNK = 1024
GRP = 128
SLICE16 = (4 * NB) // NSUB


def _score_body(x_ref, w_ref, b_ref, s_ref):
    xb = x_ref[...]
    s = jnp.sum(xb * w_ref[...], axis=1) + b_ref[0, 0]
    s_ref[...] = jnp.abs(s).reshape(B, BH * H)


def _scores_part(x, w4, bias, part):
    nblk = HPART // BH
    return pl.pallas_call(
        _score_body,
        grid=(nblk,),
        in_specs=[
            pl.BlockSpec((B, C, BH, H), lambda j: (0, 0, part * nblk + j, 0)),
            pl.BlockSpec((1, C, 1, 1), lambda j: (0, 0, 0, 0)),
            pl.BlockSpec((1, 1), lambda j: (0, 0)),
        ],
        out_specs=pl.BlockSpec((B, BH * H), lambda j: (0, j)),
        out_shape=jax.ShapeDtypeStruct((B, PPIX), jnp.float32),
    )(x, w4, bias)


def _hist_body(scores_hbm, cnt_hbm, chunk_v, idx_v, ones_v, stage_v, hist_sh):
    cid = lax.axis_index("c")
    sid = lax.axis_index("s")
    lb = sid // 4
    q = sid % 4
    batch = cid * 4 + lb

    zvec = jnp.zeros((16,), jnp.float32)

    def zero_stage(i, carry):
        stage_v[pl.ds(i * 16, 16)] = zvec
        return carry

    lax.fori_loop(0, CHUNK // 16, zero_stage, 0)

    def zero_hist(j, carry):
        pltpu.sync_copy(stage_v, hist_sh.at[pl.ds(sid * SLICE16 + j * CHUNK, CHUNK)])
        return carry

    lax.fori_loop(0, SLICE16 // CHUNK, zero_hist, 0)

    ovec = jnp.ones((16,), jnp.float32)
    for u in range(GRP // 16):
        ones_v[pl.ds(u * 16, 16)] = ovec

    plsc.subcore_barrier()

    base = q * TPIX
    off = lb * NB

    def chunk_loop(ci, carry):
        pltpu.sync_copy(scores_hbm.at[batch, pl.ds(base + ci * CHUNK, CHUNK)],
                        chunk_v)

        def grp_loop(g, c2):
            for u in range(GRP // 16):
                v = chunk_v[pl.ds(g * GRP + u * 16, 16)]
                bits = lax.bitcast_convert_type(v, jnp.int32)
                idx_v[pl.ds(u * 16, 16)] = lax.shift_right_logical(bits, 16) + off
            pltpu.sync_copy(ones_v, hist_sh.at[idx_v], add=True)
            return c2

        lax.fori_loop(0, CHUNK // GRP, grp_loop, 0)
        return carry

    lax.fori_loop(0, TPIX // CHUNK, chunk_loop, 0)

    plsc.subcore_barrier()

    pltpu.sync_copy(hist_sh.at[pl.ds(sid * SLICE16, SLICE16)],
                    cnt_hbm.at[batch, pl.ds(q * SLICE16, SLICE16)])


@functools.cache
def _hist_kernel():
    return functools.partial(
        pl.kernel,
        out_type=jax.ShapeDtypeStruct((B, NB), jnp.float32),
        mesh=plsc.VectorSubcoreMesh(core_axis_name="c", subcore_axis_name="s"),
        scratch_types=[
            pltpu.VMEM((CHUNK,), jnp.float32),
            pltpu.VMEM((GRP,), jnp.int32),
            pltpu.VMEM((GRP,), jnp.float32),
            pltpu.VMEM((CHUNK,), jnp.float32),
            pltpu.VMEM_SHARED((4 * NB,), jnp.float32),
        ],
    )(_hist_body)


def _select_body(*refs):
    cnt_refs = refs[:NSPLIT]
    s_refs = refs[NSPLIT:2 * NSPLIT]
    o_ref = refs[2 * NSPLIT]
    cnt = cnt_refs[0][...]
    for r in cnt_refs[1:]:
        cnt = cnt + r[...]
    bin_iota = lax.broadcasted_iota(jnp.int32, (B, NB), 1)
    kk = jnp.float32(K)

    def sfx_above(bidx):
        m = bin_iota > bidx
        return jnp.sum(jnp.where(m, cnt, 0.0), axis=1, keepdims=True)

    def step(_, lohi):
        lo, hi = lohi
        mid = (lo + hi) >> 1
        pred = sfx_above(mid) < kk
        return (jnp.where(pred, lo, mid + 1), jnp.where(pred, mid, hi))

    lo0 = jnp.zeros((B, 1), jnp.int32)
    hi0 = jnp.full((B, 1), NB - 1, jnp.int32)
    _, bstar = lax.fori_loop(0, 15, step, (lo0, hi0))
    c_above = sfx_above(bstar)

    s_above = jnp.zeros((B, 1), jnp.float32)
    bin_sum = jnp.zeros((B, 1), jnp.float32)
    bin_cnt = jnp.zeros((B, 1), jnp.float32)
    for r in s_refs:
        s = r[...]
        sbin = lax.shift_right_logical(
            lax.bitcast_convert_type(s, jnp.int32), 16)
        m_above = sbin > bstar
        m_bin = sbin == bstar
        s_above = s_above + jnp.sum(
            jnp.where(m_above, s, 0.0), axis=1, keepdims=True)
        bin_sum = bin_sum + jnp.sum(
            jnp.where(m_bin, s, 0.0), axis=1, keepdims=True)
        bin_cnt = bin_cnt + jnp.sum(
            jnp.where(m_bin, 1.0, 0.0), axis=1, keepdims=True)
    t_est = bin_sum / jnp.maximum(bin_cnt, 1.0)
    o_ref[...] = (s_above + (kk - c_above) * t_est) / kk


def _select(cnts, score_parts):
    return pl.pallas_call(
        _select_body,
        in_specs=(
            [pl.BlockSpec((B, NB), lambda: (0, 0)) for _ in range(NSPLIT)]
            + [pl.BlockSpec((B, PPIX), lambda: (0, 0)) for _ in range(NSPLIT)]
        ),
        out_specs=pl.BlockSpec((B, 1), lambda: (0, 0)),
        out_shape=jax.ShapeDtypeStruct((B, 1), jnp.float32),
    )(*cnts, *score_parts)


def kernel(x, W, b):
    w4 = W.reshape(1, C, 1, 1)
    bias = b.reshape(1, 1)
    cnts, parts = [], []
    for p in range(NSPLIT):
        s2 = _scores_part(x, w4, bias, p)
        parts.append(s2)
        cnts.append(_hist_kernel()(s2))
    return _select(cnts, parts)


# speedup vs baseline: 1.1414x; 1.1414x over previous
import functools

import jax
import jax.numpy as jnp
from jax import lax
from jax.experimental import pallas as pl
from jax.experimental.pallas import tpu as pltpu
from jax.experimental.pallas import tpu_sc as plsc

B = 8
C = 96
H = 384
NPIX = 384 * 384
K = int(NPIX * 0.1)
NB = 32768
BH = 8
NSPLIT = 4
NSC = 3
HPART = H // NSPLIT
PPIX = NPIX // NSPLIT
NCORES = 2
NSUB = 16
TPIX = PPIX // 4
CHUNK = 1024
GRP = 128
SLICE16 = (4 * NB) // NSUB


def _score_body(x_ref, w_ref, b_ref, s_ref):
    xb = x_ref[...]
    s = jnp.sum(xb * w_ref[...], axis=1) + b_ref[0, 0]
    s_ref[...] = jnp.abs(s).reshape(B, BH * H)


def _scores_part(x, w4, bias, part):
    nblk = HPART // BH
    return pl.pallas_call(
        _score_body,
        grid=(nblk,),
        in_specs=[
            pl.BlockSpec((B, C, BH, H), lambda j: (0, 0, part * nblk + j, 0)),
            pl.BlockSpec((1, C, 1, 1), lambda j: (0, 0, 0, 0)),
            pl.BlockSpec((1, 1), lambda j: (0, 0)),
        ],
        out_specs=pl.BlockSpec((B, BH * H), lambda j: (0, j)),
        out_shape=jax.ShapeDtypeStruct((B, PPIX), jnp.float32),
    )(x, w4, bias)


def _hist_body(scores_hbm, cnt_hbm, chunk_v, idx_v, ones_v, stage_v, hist_sh):
    cid = lax.axis_index("c")
    sid = lax.axis_index("s")
    lb = sid // 4
    q = sid % 4
    batch = cid * 4 + lb

    zvec = jnp.zeros((16,), jnp.float32)

    def zero_stage(i, carry):
        stage_v[pl.ds(i * 16, 16)] = zvec
        return carry

    lax.fori_loop(0, CHUNK // 16, zero_stage, 0)

    def zero_hist(j, carry):
        pltpu.sync_copy(stage_v, hist_sh.at[pl.ds(sid * SLICE16 + j * CHUNK, CHUNK)])
        return carry

    lax.fori_loop(0, SLICE16 // CHUNK, zero_hist, 0)

    ovec = jnp.ones((16,), jnp.float32)
    for u in range(GRP // 16):
        ones_v[pl.ds(u * 16, 16)] = ovec

    plsc.subcore_barrier()

    base = q * TPIX
    off = lb * NB

    def chunk_loop(ci, carry):
        pltpu.sync_copy(scores_hbm.at[batch, pl.ds(base + ci * CHUNK, CHUNK)],
                        chunk_v)

        def grp_loop(g, c2):
            for u in range(GRP // 16):
                v = chunk_v[pl.ds(g * GRP + u * 16, 16)]
                bits = lax.bitcast_convert_type(v, jnp.int32)
                idx_v[pl.ds(u * 16, 16)] = lax.shift_right_logical(bits, 16) + off
            pltpu.sync_copy(ones_v, hist_sh.at[idx_v], add=True)
            return c2

        lax.fori_loop(0, CHUNK // GRP, grp_loop, 0)
        return carry

    lax.fori_loop(0, TPIX // CHUNK, chunk_loop, 0)

    plsc.subcore_barrier()

    pltpu.sync_copy(hist_sh.at[pl.ds(sid * SLICE16, SLICE16)],
                    cnt_hbm.at[batch, pl.ds(q * SLICE16, SLICE16)])


@functools.cache
def _hist_kernel():
    return functools.partial(
        pl.kernel,
        out_type=jax.ShapeDtypeStruct((B, NB), jnp.float32),
        mesh=plsc.VectorSubcoreMesh(core_axis_name="c", subcore_axis_name="s"),
        scratch_types=[
            pltpu.VMEM((CHUNK,), jnp.float32),
            pltpu.VMEM((GRP,), jnp.int32),
            pltpu.VMEM((GRP,), jnp.float32),
            pltpu.VMEM((CHUNK,), jnp.float32),
            pltpu.VMEM_SHARED((4 * NB,), jnp.float32),
        ],
    )(_hist_body)


def _select_body(*refs):
    cnt_refs = refs[:NSC]
    s_refs = refs[NSC:NSC + NSPLIT]
    o_ref = refs[NSC + NSPLIT]
    cnt = cnt_refs[0][...]
    for r in cnt_refs[1:]:
        cnt = cnt + r[...]
    bin_iota = lax.broadcasted_iota(jnp.int32, (B, NB), 1)
    kk = jnp.float32(K)

    tail_bins = [
        lax.shift_right_logical(
            lax.bitcast_convert_type(r[...], jnp.int32), 16)
        for r in s_refs[NSC:]
    ]

    def sfx_above(bidx):
        m = bin_iota > bidx
        acc = jnp.sum(jnp.where(m, cnt, 0.0), axis=1, keepdims=True)
        for tb in tail_bins:
            acc = acc + jnp.sum(jnp.where(tb > bidx, 1.0, 0.0),
                                axis=1, keepdims=True)
        return acc

    def step(_, lohi):
        lo, hi = lohi
        mid = (lo + hi) >> 1
        pred = sfx_above(mid) < kk
        return (jnp.where(pred, lo, mid + 1), jnp.where(pred, mid, hi))

    lo0 = jnp.zeros((B, 1), jnp.int32)
    hi0 = jnp.full((B, 1), NB - 1, jnp.int32)
    _, bstar = lax.fori_loop(0, 15, step, (lo0, hi0))
    c_above = sfx_above(bstar)

    s_above = jnp.zeros((B, 1), jnp.float32)
    bin_sum = jnp.zeros((B, 1), jnp.float32)
    bin_cnt = jnp.zeros((B, 1), jnp.float32)
    for r in s_refs:
        s = r[...]
        sbin = lax.shift_right_logical(
            lax.bitcast_convert_type(s, jnp.int32), 16)
        m_above = sbin > bstar
        m_bin = sbin == bstar
        s_above = s_above + jnp.sum(
            jnp.where(m_above, s, 0.0), axis=1, keepdims=True)
        bin_sum = bin_sum + jnp.sum(
            jnp.where(m_bin, s, 0.0), axis=1, keepdims=True)
        bin_cnt = bin_cnt + jnp.sum(
            jnp.where(m_bin, 1.0, 0.0), axis=1, keepdims=True)
    t_est = bin_sum / jnp.maximum(bin_cnt, 1.0)
    o_ref[...] = (s_above + (kk - c_above) * t_est) / kk


def _select(cnts, score_parts):
    return pl.pallas_call(
        _select_body,
        in_specs=(
            [pl.BlockSpec((B, NB), lambda: (0, 0)) for _ in range(NSC)]
            + [pl.BlockSpec((B, PPIX), lambda: (0, 0)) for _ in range(NSPLIT)]
        ),
        out_specs=pl.BlockSpec((B, 1), lambda: (0, 0)),
        out_shape=jax.ShapeDtypeStruct((B, 1), jnp.float32),
    )(*cnts, *score_parts)


def kernel(x, W, b):
    w4 = W.reshape(1, C, 1, 1)
    bias = b.reshape(1, 1)
    cnts, parts = [], []
    for p in range(NSPLIT):
        s2 = _scores_part(x, w4, bias, p)
        parts.append(s2)
        if p < NSC:
            cnts.append(_hist_kernel()(s2))
    return _select(cnts, parts)
